# Initial kernel scaffold; baseline (speedup 1.0000x reference)
import jax
import jax.numpy as jnp
from jax.experimental import pallas as pl


def kernel(x_ids, embedding, w_ih_T, w_hh_T, b_lstm, w_cls_T, b_cls):
    raise NotImplementedError("write your pallas kernel here")


# speedup vs baseline: 1.0947x; 1.0947x over previous
import jax
import jax.numpy as jnp
from jax.experimental import pallas as pl
from jax.experimental.pallas import tpu as pltpu


def _round_up(x, m):
    return ((x + m - 1) // m) * m


def _pick_chunk(T, max_chunk=32):
    if T <= max_chunk:
        return T
    for c in range(max_chunk, 0, -1):
        if T % c == 0:
            return c
    return T


def _lstm_cls_kernel(x_ref, wih_ref, whh_ref, b_ref, wcls_ref, bcls_ref,
                     out_ref, h_ref, c_ref):
    t = pl.program_id(1)
    n_t = pl.num_programs(1)
    CHUNK, TB, Dp = x_ref.shape
    H = TB // 2

    @pl.when(t == 0)
    def _init():
        h_ref[...] = jnp.zeros_like(h_ref)
        c_ref[...] = jnp.zeros_like(c_ref)

    wih = wih_ref[...]
    whh = whh_ref[...]
    bias = b_ref[...]

    def half_step(x_t, h, c):
        gates = (jnp.dot(x_t, wih, preferred_element_type=jnp.float32)
                 + jnp.dot(h, whh, preferred_element_type=jnp.float32)
                 + bias)
        sig = jax.nn.sigmoid(gates[:, :3 * Dp])
        g_g = jnp.tanh(gates[:, 3 * Dp:])
        i_g = sig[:, :Dp]
        f_g = sig[:, Dp:2 * Dp]
        o_g = sig[:, 2 * Dp:]
        c_new = f_g * c + i_g * g_g
        h_new = (o_g * jnp.tanh(c_new)).astype(jnp.bfloat16)
        return h_new, c_new

    def step(s, carry):
        hA, cA, hB, cB = carry
        xA = x_ref[s, pl.ds(0, H), :]
        xB = x_ref[s, pl.ds(H, H), :]
        hA, cA = half_step(xA, hA, cA)
        hB, cB = half_step(xB, hB, cB)
        return hA, cA, hB, cB

    carry0 = (h_ref[pl.ds(0, H), :], c_ref[pl.ds(0, H), :],
              h_ref[pl.ds(H, H), :], c_ref[pl.ds(H, H), :])
    hA, cA, hB, cB = jax.lax.fori_loop(0, CHUNK, step, carry0, unroll=2)
    h_ref[pl.ds(0, H), :] = hA
    c_ref[pl.ds(0, H), :] = cA
    h_ref[pl.ds(H, H), :] = hB
    c_ref[pl.ds(H, H), :] = cB

    @pl.when(t == n_t - 1)
    def _finish():
        wcls = wcls_ref[...]
        bcls = bcls_ref[...]
        out_ref[pl.ds(0, H), :] = (
            jnp.dot(hA, wcls, preferred_element_type=jnp.float32) + bcls
        ).astype(out_ref.dtype)
        out_ref[pl.ds(H, H), :] = (
            jnp.dot(hB, wcls, preferred_element_type=jnp.float32) + bcls
        ).astype(out_ref.dtype)


def kernel(x_ids, embedding, w_ih_T, w_hh_T, b_lstm, w_cls_T, b_cls):
    Dp = embedding.shape[1]
    G = w_ih_T.shape[1]
    Cp = w_cls_T.shape[1]
    B, T = x_ids.shape

    if B % 256 == 0:
        TB = 256
    elif B % 16 == 0:
        TB = B
    else:
        TB = _round_up(B, 16)
    Bp = _round_up(B, TB)
    nb = Bp // TB
    CHUNK = _pick_chunk(T)
    nt = T // CHUNK

    x_tm = jnp.take(embedding, x_ids.T, axis=0)
    if Bp != B:
        x_tm = jnp.pad(x_tm, ((0, 0), (0, Bp - B), (0, 0)))

    full = lambda shape: pl.BlockSpec(shape, lambda b, t: tuple(0 for _ in shape))

    vmem_bytes = (
        2 * CHUNK * TB * Dp * 2
        + 2 * Dp * G * 2
        + G * 4 + Dp * Cp * 2 + Cp * 4
        + TB * Cp * 4
        + TB * Dp * (2 + 4)
    )
    vmem_limit = int(min(64 * 1024 * 1024, max(32 * 1024 * 1024,
                                               int(vmem_bytes * 2))))

    out = pl.pallas_call(
        _lstm_cls_kernel,
        out_shape=jax.ShapeDtypeStruct((Bp, Cp), jnp.float32),
        grid=(nb, nt),
        in_specs=[
            pl.BlockSpec((CHUNK, TB, Dp), lambda b, t: (t, b, 0)),
            full((Dp, G)),
            full((Dp, G)),
            full((1, G)),
            full((Dp, Cp)),
            full((1, Cp)),
        ],
        out_specs=pl.BlockSpec((TB, Cp), lambda b, t: (b, 0)),
        scratch_shapes=[
            pltpu.VMEM((TB, Dp), jnp.bfloat16),
            pltpu.VMEM((TB, Dp), jnp.float32),
        ],
        compiler_params=pltpu.CompilerParams(
            dimension_semantics=("parallel", "arbitrary"),
            vmem_limit_bytes=vmem_limit,
        ),
    )(x_tm, w_ih_T, w_hh_T, b_lstm, w_cls_T, b_cls)
    return out[:B, :128]
